# baseline (speedup 1.0000x reference)
import functools

import jax
import jax.numpy as jnp
from jax import lax
from jax.experimental import pallas as pl
from jax.experimental.pallas import tpu as pltpu
from jax.experimental.pallas import tpu_sc as plsc

NUM_PARENTS = 8
CHILDREN_PER_PARENT = 16
NUM_CLASSES = NUM_PARENTS + NUM_PARENTS * CHILDREN_PER_PARENT
BATCH = 16384

NC = 2
NS = 16
L = 16
NW = NC * NS
ROWS_PER_TILE = BATCH // NW
RBLK = 128
NBLK = ROWS_PER_TILE // RBLK
NCHUNK = RBLK // L

_LN2 = 0.6931471805599453


def _vlog(x):
    ib = lax.bitcast_convert_type(x, jnp.int32)
    ex = lax.shift_right_logical(ib, 23) - 127
    mb = (ib & 0x007FFFFF) | 0x3F800000
    m = lax.bitcast_convert_type(mb, jnp.float32)
    big = m > 1.4142135
    m = jnp.where(big, m * 0.5, m)
    ef = ex.astype(jnp.float32) + jnp.where(big, 1.0, 0.0)
    z = (m - 1.0) / (m + 1.0)
    z2 = z * z
    p = 2.0 + z2 * (0.66666667 + z2 * (0.4 + z2 * (0.28571429 + z2 * 0.22222222)))
    return ef * _LN2 + z * p


def _tree_reduce(op, vs):
    vs = list(vs)
    while len(vs) > 1:
        nxt = [op(vs[i], vs[i + 1]) for i in range(0, len(vs) - 1, 2)]
        if len(vs) % 2:
            nxt.append(vs[-1])
        vs = nxt
    return vs[0]


@functools.cache
def _build_sc_kernel():
    return pl.kernel(
        _sc_hier_softmax,
        out_type=[
            jax.ShapeDtypeStruct((BATCH * NUM_CLASSES,), jnp.float32),
            jax.ShapeDtypeStruct((NW, L), jnp.float32),
        ],
        mesh=plsc.VectorSubcoreMesh(core_axis_name="c", subcore_axis_name="s",
                                    num_cores=NC, num_subcores=NS),
        compiler_params=pltpu.CompilerParams(needs_layout_passes=False),
        scratch_types=[
            pltpu.VMEM((RBLK * NUM_CLASSES,), jnp.float32),
            pltpu.VMEM((RBLK * NUM_CLASSES,), jnp.float32),
            pltpu.VMEM((RBLK * NUM_CLASSES,), jnp.float32),
            pltpu.VMEM((L,), jnp.float32),
        ],
    )


def _sc_hier_softmax(pred_hbm, targ_hbm, out_hbm, part_hbm,
                     pred_vm, targ_vm, out_vm, acc_vm):
    wid = lax.axis_index("s") * NC + lax.axis_index("c")
    base_row = wid * ROWS_PER_TILE
    lanes = lax.iota(jnp.int32, L)

    def chunk_body(ch, acc):
        rowbase = (ch * L + lanes) * NUM_CLASSES

        idxs = [rowbase + c for c in range(NUM_PARENTS)]
        pv = [plsc.load_gather(pred_vm, [idx]) for idx in idxs]
        tv = [plsc.load_gather(targ_vm, [idx]) for idx in idxs]
        m = _tree_reduce(jnp.maximum, pv)
        s_vt = _tree_reduce(jnp.add, [v * t for v, t in zip(pv, tv)])
        s_t = _tree_reduce(jnp.add, tv)
        ev = [jnp.exp(v - m) for v in pv]
        s = _tree_reduce(jnp.add, ev)
        log_z = m + _vlog(s)
        plp = [v - log_z for v in pv]
        acc = acc + (s_vt - log_z * s_t)
        rcp = 1.0 / s
        for c in range(NUM_PARENTS):
            plsc.store_scatter(out_vm, [idxs[c]], ev[c] * rcp)

        for g in range(NUM_PARENTS):
            col0 = NUM_PARENTS + g * CHILDREN_PER_PARENT
            idxs = [rowbase + (col0 + c) for c in range(CHILDREN_PER_PARENT)]
            mg = None
            s_vt = None
            s_t = None
            for c in range(CHILDREN_PER_PARENT):
                v = plsc.load_gather(pred_vm, [idxs[c]])
                t = plsc.load_gather(targ_vm, [idxs[c]])
                mg = v if mg is None else jnp.maximum(mg, v)
                s_vt = v * t if s_vt is None else s_vt + v * t
                s_t = t if s_t is None else s_t + t
            eg = []
            for c in range(CHILDREN_PER_PARENT):
                v = plsc.load_gather(pred_vm, [idxs[c]])
                eg.append(jnp.exp(v - mg))
            sg = _tree_reduce(jnp.add, eg)
            base = mg + _vlog(sg) - plp[g]
            acc = acc + (s_vt - base * s_t)
            pf = jnp.exp(plp[g]) / sg
            for c in range(CHILDREN_PER_PARENT):
                plsc.store_scatter(out_vm, [idxs[c]], eg[c] * pf)
        return acc

    def block_body(blk, acc):
        off = (base_row + blk * RBLK) * NUM_CLASSES
        pltpu.sync_copy(pred_hbm.at[pl.ds(off, RBLK * NUM_CLASSES)], pred_vm)
        pltpu.sync_copy(targ_hbm.at[pl.ds(off, RBLK * NUM_CLASSES)], targ_vm)
        acc = lax.fori_loop(0, NCHUNK, chunk_body, acc)
        pltpu.sync_copy(out_vm, out_hbm.at[pl.ds(off, RBLK * NUM_CLASSES)])
        return acc

    acc = lax.fori_loop(0, NBLK, block_body, jnp.zeros((L,), jnp.float32))
    acc_vm[...] = acc
    pltpu.sync_copy(acc_vm, part_hbm.at[wid])


def kernel(pred, target, _):
    out_flat, parts = _build_sc_kernel()(pred.reshape(-1), target.reshape(-1))
    loss = -(parts.sum() / BATCH)
    return (loss, out_flat.reshape(BATCH, NUM_CLASSES))


# speedup vs baseline: 1.4398x; 1.4398x over previous
import functools

import jax
import jax.numpy as jnp
from jax import lax
from jax.experimental import pallas as pl
from jax.experimental.pallas import tpu as pltpu
from jax.experimental.pallas import tpu_sc as plsc

NUM_PARENTS = 8
CHILDREN_PER_PARENT = 16
NUM_CLASSES = NUM_PARENTS + NUM_PARENTS * CHILDREN_PER_PARENT
BATCH = 16384

NC = 2
NS = 16
L = 16
NW = NC * NS
ROWS_PER_TILE = BATCH // NW
RBLK = 128
NBLK = ROWS_PER_TILE // RBLK
NCHUNK = RBLK // L

_LN2 = 0.6931471805599453


def _vlog(x):
    ib = lax.bitcast_convert_type(x, jnp.int32)
    ex = lax.shift_right_logical(ib, 23) - 127
    mb = (ib & 0x007FFFFF) | 0x3F800000
    m = lax.bitcast_convert_type(mb, jnp.float32)
    big = m > 1.4142135
    m = jnp.where(big, m * 0.5, m)
    ef = ex.astype(jnp.float32) + jnp.where(big, 1.0, 0.0)
    z = (m - 1.0) / (m + 1.0)
    z2 = z * z
    p = 2.0 + z2 * (0.66666667 + z2 * (0.4 + z2 * (0.28571429 + z2 * 0.22222222)))
    return ef * _LN2 + z * p


def _tree_reduce(op, vs):
    vs = list(vs)
    while len(vs) > 1:
        nxt = [op(vs[i], vs[i + 1]) for i in range(0, len(vs) - 1, 2)]
        if len(vs) % 2:
            nxt.append(vs[-1])
        vs = nxt
    return vs[0]


@functools.cache
def _build_sc_kernel():
    return pl.kernel(
        _sc_hier_softmax,
        out_type=[
            jax.ShapeDtypeStruct((BATCH * NUM_CLASSES,), jnp.float32),
            jax.ShapeDtypeStruct((NW, L), jnp.float32),
        ],
        mesh=plsc.VectorSubcoreMesh(core_axis_name="c", subcore_axis_name="s",
                                    num_cores=NC, num_subcores=NS),
        compiler_params=pltpu.CompilerParams(needs_layout_passes=False),
        scratch_types=[
            pltpu.VMEM((RBLK * NUM_CLASSES,), jnp.float32),
            pltpu.VMEM((RBLK * NUM_CLASSES,), jnp.float32),
            pltpu.VMEM((RBLK * NUM_CLASSES,), jnp.float32),
            pltpu.VMEM((NUM_PARENTS * L,), jnp.float32),
            pltpu.VMEM((L,), jnp.float32),
        ],
    )


def _sc_hier_softmax(pred_hbm, targ_hbm, out_hbm, part_hbm,
                     pred_vm, targ_vm, out_vm, plp_vm, acc_vm):
    wid = lax.axis_index("s") * NC + lax.axis_index("c")
    base_row = wid * ROWS_PER_TILE
    lanes = lax.iota(jnp.int32, L)

    def chunk_body(ch, acc):
        rowbase = (ch * L + lanes) * NUM_CLASSES

        idxs = [rowbase + c for c in range(NUM_PARENTS)]
        pv = [plsc.load_gather(pred_vm, [idx]) for idx in idxs]
        tv = [plsc.load_gather(targ_vm, [idx]) for idx in idxs]
        m = _tree_reduce(jnp.maximum, pv)
        s_vt = _tree_reduce(jnp.add, [v * t for v, t in zip(pv, tv)])
        s_t = _tree_reduce(jnp.add, tv)
        ev = [jnp.exp(v - m) for v in pv]
        s = _tree_reduce(jnp.add, ev)
        log_z = m + _vlog(s)
        plp = [v - log_z for v in pv]
        acc = acc + (s_vt - log_z * s_t)
        rcp = 1.0 / s
        for c in range(NUM_PARENTS):
            plsc.store_scatter(out_vm, [idxs[c]], ev[c] * rcp)

        for c in range(NUM_PARENTS):
            plp_vm[pl.ds(c * L, L)] = plp[c]

        def group_body(g, acc):
            col0 = NUM_PARENTS + g * CHILDREN_PER_PARENT
            cb = rowbase + col0
            idxs = [cb + c for c in range(CHILDREN_PER_PARENT)]
            plp_g = plp_vm[pl.ds(g * L, L)]
            vs = [plsc.load_gather(pred_vm, [idx]) for idx in idxs]
            ts = [plsc.load_gather(targ_vm, [idx]) for idx in idxs]
            mg = _tree_reduce(jnp.maximum, vs)
            s_vt = _tree_reduce(jnp.add, [v * t for v, t in zip(vs, ts)])
            s_t = _tree_reduce(jnp.add, ts)
            eg = [jnp.exp(v - mg) for v in vs]
            sg = _tree_reduce(jnp.add, eg)
            base = mg + _vlog(sg) - plp_g
            acc = acc + (s_vt - base * s_t)
            pf = jnp.exp(plp_g) / sg
            for c in range(CHILDREN_PER_PARENT):
                plsc.store_scatter(out_vm, [idxs[c]], eg[c] * pf)
            return acc

        return lax.fori_loop(0, NUM_PARENTS, group_body, acc)

    def block_body(blk, acc):
        off = (base_row + blk * RBLK) * NUM_CLASSES
        pltpu.sync_copy(pred_hbm.at[pl.ds(off, RBLK * NUM_CLASSES)], pred_vm)
        pltpu.sync_copy(targ_hbm.at[pl.ds(off, RBLK * NUM_CLASSES)], targ_vm)
        acc = lax.fori_loop(0, NCHUNK, chunk_body, acc)
        pltpu.sync_copy(out_vm, out_hbm.at[pl.ds(off, RBLK * NUM_CLASSES)])
        return acc

    acc = lax.fori_loop(0, NBLK, block_body, jnp.zeros((L,), jnp.float32))
    acc_vm[...] = acc
    pltpu.sync_copy(acc_vm, part_hbm.at[wid])


def kernel(pred, target, _):
    out_flat, parts = _build_sc_kernel()(pred.reshape(-1), target.reshape(-1))
    loss = -(parts.sum() / BATCH)
    return (loss, out_flat.reshape(BATCH, NUM_CLASSES))
